# baseline (speedup 1.0000x reference)
import functools

import jax
import jax.numpy as jnp
from jax.experimental import pallas as pl
from jax.experimental.pallas import tpu as pltpu

T, D_MODEL, D_FF, E, TOP_K = 2048, 768, 2048, 8, 2
EPAD = 128
NEG = -1e30


def _routing_body(x_ref, wg_ref, steer_ref, c_ref):
    logits = jnp.dot(x_ref[...], wg_ref[...], preferred_element_type=jnp.float32)
    logits = logits + steer_ref[...]
    lane = jax.lax.broadcasted_iota(jnp.int32, (T, EPAD), 1)
    m1 = jnp.max(logits, axis=1, keepdims=True)
    i1 = jnp.min(jnp.where(logits == m1, lane, EPAD), axis=1, keepdims=True)
    masked = jnp.where(lane == i1, NEG, logits)
    m2 = jnp.max(masked, axis=1, keepdims=True)
    i2 = jnp.min(jnp.where(masked == m2, lane, EPAD), axis=1, keepdims=True)
    w0 = 1.0 / (1.0 + jnp.exp(m2 - m1))
    w1 = 1.0 - w0
    c_ref[...] = jnp.where(lane == i1, w0, 0.0) + jnp.where(lane == i2, w1, 0.0)


def _ffn_body(c_ref, x_ref, w1_ref, w3_ref, w2_ref, out_ref):
    e = pl.program_id(0)
    f = pl.program_id(1)
    xb = x_ref[...]
    a = jnp.dot(xb, w1_ref[0], preferred_element_type=jnp.float32)
    b = jnp.dot(xb, w3_ref[0], preferred_element_type=jnp.float32)
    h = (a * jax.nn.sigmoid(a)) * b
    p = jnp.dot(h.astype(jnp.bfloat16), w2_ref[0], preferred_element_type=jnp.float32)
    lane = jax.lax.broadcasted_iota(jnp.int32, (T, EPAD), 1)
    c_col = jnp.sum(jnp.where(lane == e, c_ref[...], 0.0), axis=1, keepdims=True)
    contrib = c_col * p

    @pl.when((e == 0) & (f == 0))
    def _init():
        out_ref[...] = contrib

    @pl.when((e > 0) | (f > 0))
    def _acc():
        out_ref[...] += contrib


def kernel(hidden_states, Wg, steer_vector, W1, W3, W2):
    x = hidden_states
    wg_pad = jnp.zeros((D_MODEL, EPAD), jnp.float32).at[:, :E].set(Wg)
    steer_pad = jnp.full((1, EPAD), NEG, jnp.float32).at[0, :E].set(steer_vector)

    c = pl.pallas_call(
        _routing_body,
        out_shape=jax.ShapeDtypeStruct((T, EPAD), jnp.float32),
    )(x, wg_pad, steer_pad)

    xb16 = x.astype(jnp.bfloat16)
    w1b = W1.astype(jnp.bfloat16)
    w3b = W3.astype(jnp.bfloat16)
    w2b = W2.astype(jnp.bfloat16)
    FB = 512
    NF = D_FF // FB
    out = pl.pallas_call(
        _ffn_body,
        grid=(E, NF),
        in_specs=[
            pl.BlockSpec((T, EPAD), lambda e, f: (0, 0)),
            pl.BlockSpec((T, D_MODEL), lambda e, f: (0, 0)),
            pl.BlockSpec((1, D_MODEL, FB), lambda e, f: (e, 0, f)),
            pl.BlockSpec((1, D_MODEL, FB), lambda e, f: (e, 0, f)),
            pl.BlockSpec((1, FB, D_MODEL), lambda e, f: (e, f, 0)),
        ],
        out_specs=pl.BlockSpec((T, D_MODEL), lambda e, f: (0, 0)),
        out_shape=jax.ShapeDtypeStruct((T, D_MODEL), jnp.float32),
        compiler_params=pltpu.CompilerParams(
            dimension_semantics=("arbitrary", "arbitrary"),
        ),
    )(c, xb16, w1b, w3b, w2b)
    return out


# speedup vs baseline: 1.1662x; 1.1662x over previous
import functools

import jax
import jax.numpy as jnp
from jax import lax
from jax.experimental import pallas as pl
from jax.experimental.pallas import tpu as pltpu
from jax.experimental.pallas import tpu_sc as plsc

T, D_MODEL, D_FF, E, TOP_K = 2048, 768, 2048, 8, 2
EPAD = 128
NEG = -1e30
B = 256
NB = 24
S = NB * B
FB = 512
NF = D_FF // FB
NW = 32
CH = T // NW
LANES = 16
CA = 256
NCA = T // CA


def _k1a_body(x_ref, wg_ref, steer_ref,
              a1_ref, a2_ref, wb0_ref, wb1_ref, c1_ref, c2_ref,
              p1_ref, p2_ref):
    logits = jnp.dot(x_ref[...], wg_ref[...], preferred_element_type=jnp.float32)
    logits = logits + steer_ref[...]
    lane = lax.broadcasted_iota(jnp.int32, (CA, EPAD), 1)
    m1 = jnp.max(logits, axis=1, keepdims=True)
    i1 = jnp.min(jnp.where(logits == m1, lane, EPAD), axis=1, keepdims=True)
    masked = jnp.where(lane == i1, NEG, logits)
    m2 = jnp.max(masked, axis=1, keepdims=True)
    i2 = jnp.min(jnp.where(masked == m2, lane, EPAD), axis=1, keepdims=True)
    w0 = 1.0 / (1.0 + jnp.exp(m2 - m1))
    w1 = 1.0 - w0
    a1_ref[...] = i1
    a2_ref[...] = i2
    wb0_ref[...] = w0 + jnp.zeros((CA, LANES), jnp.float32)
    wb1_ref[...] = w1 + jnp.zeros((CA, LANES), jnp.float32)

    oh1 = (lane == i1).astype(jnp.float32)
    oh2 = (lane == i2).astype(jnp.float32)
    c1_ref[...] = jnp.sum(oh1, axis=0, keepdims=True).reshape(1, 1, EPAD)
    c2_ref[...] = jnp.sum(oh2, axis=0, keepdims=True).reshape(1, 1, EPAD)
    tri = (lax.broadcasted_iota(jnp.int32, (CA, CA), 1)
           < lax.broadcasted_iota(jnp.int32, (CA, CA), 0)).astype(jnp.float32)
    pre1 = jnp.dot(tri, oh1, preferred_element_type=jnp.float32)
    pre2 = jnp.dot(tri, oh2, preferred_element_type=jnp.float32)
    p1_ref[...] = jnp.sum(jnp.where(lane == i1, pre1, 0.0), axis=1, keepdims=True)
    p2_ref[...] = jnp.sum(jnp.where(lane == i2, pre2, 0.0), axis=1, keepdims=True)


def _k1c_body(a1_ref, a2_ref, p1_ref, p2_ref, c1_ref, c2_ref,
              s0_ref, s1_ref, bexp_ref):
    cnt1 = c1_ref[...]
    cnt2 = c2_ref[...]
    cntb = cnt1 + cnt2
    tri = (lax.broadcasted_iota(jnp.int32, (NCA, NCA), 0)
           > lax.broadcasted_iota(jnp.int32, (NCA, NCA), 1)).astype(jnp.float32)
    excl = jnp.dot(tri, cntb, preferred_element_type=jnp.float32)
    counts = jnp.sum(cntb, axis=0, keepdims=True)
    padded = jnp.ceil(counts / B) * B
    up = (lax.broadcasted_iota(jnp.int32, (EPAD, EPAD), 0)
          < lax.broadcasted_iota(jnp.int32, (EPAD, EPAD), 1)).astype(jnp.float32)
    off = jnp.dot(padded, up, preferred_element_type=jnp.float32)
    base0 = off + excl
    base1 = base0 + cnt1
    chunk_id = lax.broadcasted_iota(jnp.int32, (T, EPAD), 0) // CA
    b0tok = jnp.zeros((T, EPAD), jnp.float32)
    b1tok = jnp.zeros((T, EPAD), jnp.float32)
    for c in range(NCA):
        b0tok = jnp.where(chunk_id == c, base0[c:c + 1, :], b0tok)
        b1tok = jnp.where(chunk_id == c, base1[c:c + 1, :], b1tok)
    lane = lax.broadcasted_iota(jnp.int32, (T, EPAD), 1)
    sel0 = jnp.sum(jnp.where(lane == a1_ref[...], b0tok, 0.0), axis=1,
                   keepdims=True)
    sel1 = jnp.sum(jnp.where(lane == a2_ref[...], b1tok, 0.0), axis=1,
                   keepdims=True)
    s0_ref[...] = (sel0 + p1_ref[...]).astype(jnp.int32)
    s1_ref[...] = (sel1 + p2_ref[...]).astype(jnp.int32)
    off_end = off + padded
    bb = (lax.broadcasted_iota(jnp.int32, (NB, EPAD), 0) * B).astype(jnp.float32)
    lane_b = lax.broadcasted_iota(jnp.int32, (NB, EPAD), 1)
    before = ((off_end <= bb) & (lane_b < E)).astype(jnp.int32)
    bexp_ref[...] = jnp.minimum(jnp.sum(before, axis=1, keepdims=True), E - 1)


def _k3_body(bexp_ref, xs_ref, w1_ref, w3_ref, w2_ref, y_ref):
    f = pl.program_id(1)
    xb = xs_ref[...]
    a = jnp.dot(xb, w1_ref[0], preferred_element_type=jnp.float32)
    b = jnp.dot(xb, w3_ref[0], preferred_element_type=jnp.float32)
    h = (a * jax.nn.sigmoid(a)) * b
    p = jnp.dot(h, w2_ref[0], preferred_element_type=jnp.float32)

    @pl.when(f == 0)
    def _init():
        y_ref[...] = p

    @pl.when(f > 0)
    def _acc():
        y_ref[...] += p


def _k2_sc(s0f, s1f, x):
    @functools.partial(
        pl.kernel,
        mesh=plsc.VectorSubcoreMesh(core_axis_name="c", subcore_axis_name="s"),
        out_type=jax.ShapeDtypeStruct((S, D_MODEL), jnp.float32),
        scratch_types=[
            pltpu.VMEM((CH,), jnp.int32),
            pltpu.VMEM((CH,), jnp.int32),
            pltpu.VMEM((CH, D_MODEL), jnp.float32),
            pltpu.SemaphoreType.DMA,
        ],
    )
    def k2(s0_hbm, s1_hbm, x_hbm, xs_hbm, s0v, s1v, xch, sem):
        nc = lax.axis_size("c")
        wid = lax.axis_index("s") * nc + lax.axis_index("c")
        tb = wid * CH
        pltpu.sync_copy(s0_hbm.at[pl.ds(tb, CH)], s0v)
        pltpu.sync_copy(s1_hbm.at[pl.ds(tb, CH)], s1v)
        pltpu.sync_copy(x_hbm.at[pl.ds(tb, CH)], xch)
        pltpu.async_copy(xch, xs_hbm.at[s0v], sem).wait()
        pltpu.async_copy(xch, xs_hbm.at[s1v], sem).wait()

    return k2(s0f, s1f, x)


def _k4_sc(y, s0, s1, wb0, wb1):
    @functools.partial(
        pl.kernel,
        mesh=plsc.VectorSubcoreMesh(core_axis_name="c", subcore_axis_name="s"),
        out_type=jax.ShapeDtypeStruct((T, D_MODEL), jnp.float32),
        scratch_types=[
            pltpu.VMEM((CH,), jnp.int32),
            pltpu.VMEM((CH,), jnp.int32),
            pltpu.VMEM((CH, LANES), jnp.float32),
            pltpu.VMEM((CH, LANES), jnp.float32),
            pltpu.VMEM((CH, D_MODEL), jnp.float32),
            pltpu.VMEM((CH, D_MODEL), jnp.float32),
            pltpu.SemaphoreType.DMA,
        ],
    )
    def k4(y_hbm, s0_hbm, s1_hbm, wb0_hbm, wb1_hbm, out_hbm,
           s0v, s1v, w0v, w1v, r0, r1, sem):
        nc = lax.axis_size("c")
        wid = lax.axis_index("s") * nc + lax.axis_index("c")
        tb = wid * CH
        pltpu.sync_copy(s0_hbm.at[pl.ds(tb, CH)], s0v)
        pltpu.sync_copy(s1_hbm.at[pl.ds(tb, CH)], s1v)
        pltpu.sync_copy(wb0_hbm.at[pl.ds(tb, CH)], w0v)
        pltpu.sync_copy(wb1_hbm.at[pl.ds(tb, CH)], w1v)
        pltpu.async_copy(y_hbm.at[s0v], r0, sem).wait()
        pltpu.async_copy(y_hbm.at[s1v], r1, sem).wait()

        def body(i, carry):
            w0row = w0v[i, :]
            w1row = w1v[i, :]
            for j in range(D_MODEL // LANES):
                sl = pl.ds(LANES * j, LANES)
                r0[i, sl] = w0row * r0[i, sl] + w1row * r1[i, sl]
            return carry

        lax.fori_loop(0, CH, body, 0)
        pltpu.sync_copy(r0, out_hbm.at[pl.ds(tb, CH)])

    return k4(y, s0, s1, wb0, wb1)


def kernel(hidden_states, Wg, steer_vector, W1, W3, W2):
    x = hidden_states
    wg_pad = jnp.zeros((D_MODEL, EPAD), jnp.float32).at[:, :E].set(Wg)
    steer_pad = jnp.full((1, EPAD), NEG, jnp.float32).at[0, :E].set(steer_vector)

    a1, a2, wb0, wb1, c1, c2, p1, p2 = pl.pallas_call(
        _k1a_body,
        grid=(NCA,),
        in_specs=[
            pl.BlockSpec((CA, D_MODEL), lambda c: (c, 0)),
            pl.BlockSpec((D_MODEL, EPAD), lambda c: (0, 0)),
            pl.BlockSpec((1, EPAD), lambda c: (0, 0)),
        ],
        out_specs=[
            pl.BlockSpec((CA, 1), lambda c: (c, 0)),
            pl.BlockSpec((CA, 1), lambda c: (c, 0)),
            pl.BlockSpec((CA, LANES), lambda c: (c, 0)),
            pl.BlockSpec((CA, LANES), lambda c: (c, 0)),
            pl.BlockSpec((1, 1, EPAD), lambda c: (c, 0, 0)),
            pl.BlockSpec((1, 1, EPAD), lambda c: (c, 0, 0)),
            pl.BlockSpec((CA, 1), lambda c: (c, 0)),
            pl.BlockSpec((CA, 1), lambda c: (c, 0)),
        ],
        out_shape=[
            jax.ShapeDtypeStruct((T, 1), jnp.int32),
            jax.ShapeDtypeStruct((T, 1), jnp.int32),
            jax.ShapeDtypeStruct((T, LANES), jnp.float32),
            jax.ShapeDtypeStruct((T, LANES), jnp.float32),
            jax.ShapeDtypeStruct((NCA, 1, EPAD), jnp.float32),
            jax.ShapeDtypeStruct((NCA, 1, EPAD), jnp.float32),
            jax.ShapeDtypeStruct((T, 1), jnp.float32),
            jax.ShapeDtypeStruct((T, 1), jnp.float32),
        ],
    )(x, wg_pad, steer_pad)

    s0, s1, bexp = pl.pallas_call(
        _k1c_body,
        out_shape=[
            jax.ShapeDtypeStruct((T, 1), jnp.int32),
            jax.ShapeDtypeStruct((T, 1), jnp.int32),
            jax.ShapeDtypeStruct((NB, 1), jnp.int32),
        ],
    )(a1, a2, p1, p2, c1.reshape(NCA, EPAD), c2.reshape(NCA, EPAD))

    xs = _k2_sc(s0.reshape(T), s1.reshape(T), x)

    y = pl.pallas_call(
        _k3_body,
        grid_spec=pltpu.PrefetchScalarGridSpec(
            num_scalar_prefetch=1,
            grid=(NB, NF),
            in_specs=[
                pl.BlockSpec((B, D_MODEL), lambda b, f, be: (b, 0)),
                pl.BlockSpec((1, D_MODEL, FB), lambda b, f, be: (be[b], 0, f)),
                pl.BlockSpec((1, D_MODEL, FB), lambda b, f, be: (be[b], 0, f)),
                pl.BlockSpec((1, FB, D_MODEL), lambda b, f, be: (be[b], f, 0)),
            ],
            out_specs=pl.BlockSpec((B, D_MODEL), lambda b, f, be: (b, 0)),
        ),
        out_shape=jax.ShapeDtypeStruct((S, D_MODEL), jnp.float32),
        compiler_params=pltpu.CompilerParams(
            dimension_semantics=("arbitrary", "arbitrary"),
        ),
    )(bexp.reshape(NB), xs, W1, W3, W2)

    return _k4_sc(y, s0.reshape(T), s1.reshape(T), wb0, wb1)


# speedup vs baseline: 1.7092x; 1.4656x over previous
import functools

import jax
import jax.numpy as jnp
from jax import lax
from jax.experimental import pallas as pl
from jax.experimental.pallas import tpu as pltpu
from jax.experimental.pallas import tpu_sc as plsc

T, D_MODEL, D_FF, E, TOP_K = 2048, 768, 2048, 8, 2
EPAD = 128
NEG = -1e30
B = 256
NB = 24
S = NB * B
FB = 512
NF = D_FF // FB
NW = 32
CH = T // NW
LANES = 16
CA = 256
NCA = T // CA


def _k1a_body(x_ref, wg_ref, steer_ref,
              a1_ref, a2_ref, wb0_ref, wb1_ref, c1_ref, c2_ref,
              p1_ref, p2_ref):
    logits = jnp.dot(x_ref[...], wg_ref[...], preferred_element_type=jnp.float32)
    logits = logits + steer_ref[...]
    lane = lax.broadcasted_iota(jnp.int32, (CA, EPAD), 1)
    m1 = jnp.max(logits, axis=1, keepdims=True)
    i1 = jnp.min(jnp.where(logits == m1, lane, EPAD), axis=1, keepdims=True)
    masked = jnp.where(lane == i1, NEG, logits)
    m2 = jnp.max(masked, axis=1, keepdims=True)
    i2 = jnp.min(jnp.where(masked == m2, lane, EPAD), axis=1, keepdims=True)
    w0 = 1.0 / (1.0 + jnp.exp(m2 - m1))
    w1 = 1.0 - w0
    a1_ref[...] = i1
    a2_ref[...] = i2
    wb0_ref[...] = w0 + jnp.zeros((CA, LANES), jnp.float32)
    wb1_ref[...] = w1 + jnp.zeros((CA, LANES), jnp.float32)

    oh1 = (lane == i1).astype(jnp.float32)
    oh2 = (lane == i2).astype(jnp.float32)
    c1_ref[...] = jnp.sum(oh1, axis=0, keepdims=True).reshape(1, 1, EPAD)
    c2_ref[...] = jnp.sum(oh2, axis=0, keepdims=True).reshape(1, 1, EPAD)
    tri = (lax.broadcasted_iota(jnp.int32, (CA, CA), 1)
           < lax.broadcasted_iota(jnp.int32, (CA, CA), 0)).astype(jnp.float32)
    pre1 = jnp.dot(tri, oh1, preferred_element_type=jnp.float32)
    pre2 = jnp.dot(tri, oh2, preferred_element_type=jnp.float32)
    p1_ref[...] = jnp.sum(jnp.where(lane == i1, pre1, 0.0), axis=1, keepdims=True)
    p2_ref[...] = jnp.sum(jnp.where(lane == i2, pre2, 0.0), axis=1, keepdims=True)


def _k1c_body(a1_ref, a2_ref, p1_ref, p2_ref, c1_ref, c2_ref,
              s0_ref, s1_ref, bexp_ref):
    cnt1 = c1_ref[...]
    cnt2 = c2_ref[...]
    cntb = cnt1 + cnt2
    tri = (lax.broadcasted_iota(jnp.int32, (NCA, NCA), 0)
           > lax.broadcasted_iota(jnp.int32, (NCA, NCA), 1)).astype(jnp.float32)
    excl = jnp.dot(tri, cntb, preferred_element_type=jnp.float32)
    counts = jnp.sum(cntb, axis=0, keepdims=True)
    padded = jnp.ceil(counts / B) * B
    up = (lax.broadcasted_iota(jnp.int32, (EPAD, EPAD), 0)
          < lax.broadcasted_iota(jnp.int32, (EPAD, EPAD), 1)).astype(jnp.float32)
    off = jnp.dot(padded, up, preferred_element_type=jnp.float32)
    base0 = off + excl
    base1 = base0 + cnt1
    chunk_id = lax.broadcasted_iota(jnp.int32, (T, EPAD), 0) // CA
    b0tok = jnp.zeros((T, EPAD), jnp.float32)
    b1tok = jnp.zeros((T, EPAD), jnp.float32)
    for c in range(NCA):
        b0tok = jnp.where(chunk_id == c, base0[c:c + 1, :], b0tok)
        b1tok = jnp.where(chunk_id == c, base1[c:c + 1, :], b1tok)
    lane = lax.broadcasted_iota(jnp.int32, (T, EPAD), 1)
    sel0 = jnp.sum(jnp.where(lane == a1_ref[...], b0tok, 0.0), axis=1,
                   keepdims=True)
    sel1 = jnp.sum(jnp.where(lane == a2_ref[...], b1tok, 0.0), axis=1,
                   keepdims=True)
    s0_ref[...] = (sel0 + p1_ref[...]).astype(jnp.int32)
    s1_ref[...] = (sel1 + p2_ref[...]).astype(jnp.int32)
    off_end = off + padded
    bb = (lax.broadcasted_iota(jnp.int32, (NB + 1, EPAD), 0) * B).astype(jnp.float32)
    lane_b = lax.broadcasted_iota(jnp.int32, (NB + 1, EPAD), 1)
    before = ((off_end <= bb) & (lane_b < E)).astype(jnp.int32)
    raw = jnp.sum(before, axis=1, keepdims=True)
    lastexp = jnp.max(
        jnp.where((counts > 0) & (lane_b[:1] < E), lane_b[:1], 0),
        axis=1, keepdims=True)
    total_rows = jnp.sum(padded * (lane_b[:1] < E), axis=1, keepdims=True)
    nblocks = (total_rows / B).astype(jnp.int32)
    row0 = (lax.broadcasted_iota(jnp.int32, (NB + 1, 1), 0) * B)
    bexp = jnp.where(row0 < total_rows.astype(jnp.int32), raw, lastexp)
    is_last = lax.broadcasted_iota(jnp.int32, (NB + 1, 1), 0) == NB
    bexp_ref[...] = jnp.where(is_last, nblocks, bexp)


def _k3_body(bexp_ref, xs_ref, w1_ref, w3_ref, w2_ref, y_ref):
    @pl.when(pl.program_id(0) < bexp_ref[NB])
    def _compute():
        xb = xs_ref[...]
        a = jnp.dot(xb, w1_ref[0], preferred_element_type=jnp.float32)
        b = jnp.dot(xb, w3_ref[0], preferred_element_type=jnp.float32)
        h = (a * jax.nn.sigmoid(a)) * b
        y_ref[...] = jnp.dot(h, w2_ref[0], preferred_element_type=jnp.float32)


def _k2_sc(s0f, s1f, x):
    @functools.partial(
        pl.kernel,
        mesh=plsc.VectorSubcoreMesh(core_axis_name="c", subcore_axis_name="s"),
        out_type=jax.ShapeDtypeStruct((S, D_MODEL), jnp.float32),
        scratch_types=[
            pltpu.VMEM((CH,), jnp.int32),
            pltpu.VMEM((CH,), jnp.int32),
            pltpu.VMEM((CH, D_MODEL), jnp.float32),
            pltpu.SemaphoreType.DMA,
        ],
    )
    def k2(s0_hbm, s1_hbm, x_hbm, xs_hbm, s0v, s1v, xch, sem):
        nc = lax.axis_size("c")
        wid = lax.axis_index("s") * nc + lax.axis_index("c")
        tb = wid * CH
        pltpu.sync_copy(s0_hbm.at[pl.ds(tb, CH)], s0v)
        pltpu.sync_copy(s1_hbm.at[pl.ds(tb, CH)], s1v)
        pltpu.sync_copy(x_hbm.at[pl.ds(tb, CH)], xch)
        pltpu.async_copy(xch, xs_hbm.at[s0v], sem).wait()
        pltpu.async_copy(xch, xs_hbm.at[s1v], sem).wait()

    return k2(s0f, s1f, x)


def _k4_sc(y, s0, s1, wb0, wb1):
    @functools.partial(
        pl.kernel,
        mesh=plsc.VectorSubcoreMesh(core_axis_name="c", subcore_axis_name="s"),
        out_type=jax.ShapeDtypeStruct((T, D_MODEL), jnp.float32),
        scratch_types=[
            pltpu.VMEM((CH,), jnp.int32),
            pltpu.VMEM((CH,), jnp.int32),
            pltpu.VMEM((CH, LANES), jnp.float32),
            pltpu.VMEM((CH, LANES), jnp.float32),
            pltpu.VMEM((CH, D_MODEL), jnp.float32),
            pltpu.VMEM((CH, D_MODEL), jnp.float32),
            pltpu.SemaphoreType.DMA,
        ],
    )
    def k4(y_hbm, s0_hbm, s1_hbm, wb0_hbm, wb1_hbm, out_hbm,
           s0v, s1v, w0v, w1v, r0, r1, sem):
        nc = lax.axis_size("c")
        wid = lax.axis_index("s") * nc + lax.axis_index("c")
        tb = wid * CH
        pltpu.sync_copy(s0_hbm.at[pl.ds(tb, CH)], s0v)
        pltpu.sync_copy(s1_hbm.at[pl.ds(tb, CH)], s1v)
        pltpu.sync_copy(wb0_hbm.at[pl.ds(tb, CH)], w0v)
        pltpu.sync_copy(wb1_hbm.at[pl.ds(tb, CH)], w1v)
        pltpu.async_copy(y_hbm.at[s0v], r0, sem).wait()
        pltpu.async_copy(y_hbm.at[s1v], r1, sem).wait()

        def body(i, carry):
            w0row = w0v[i, :]
            w1row = w1v[i, :]
            for j in range(D_MODEL // LANES):
                sl = pl.ds(LANES * j, LANES)
                r0[i, sl] = w0row * r0[i, sl] + w1row * r1[i, sl]
            return carry

        lax.fori_loop(0, CH, body, 0)
        pltpu.sync_copy(r0, out_hbm.at[pl.ds(tb, CH)])

    return k4(y, s0, s1, wb0, wb1)


def kernel(hidden_states, Wg, steer_vector, W1, W3, W2):
    x = hidden_states
    wg_pad = jnp.zeros((D_MODEL, EPAD), jnp.float32).at[:, :E].set(Wg)
    steer_pad = jnp.full((1, EPAD), NEG, jnp.float32).at[0, :E].set(steer_vector)

    a1, a2, wb0, wb1, c1, c2, p1, p2 = pl.pallas_call(
        _k1a_body,
        grid=(NCA,),
        in_specs=[
            pl.BlockSpec((CA, D_MODEL), lambda c: (c, 0)),
            pl.BlockSpec((D_MODEL, EPAD), lambda c: (0, 0)),
            pl.BlockSpec((1, EPAD), lambda c: (0, 0)),
        ],
        out_specs=[
            pl.BlockSpec((CA, 1), lambda c: (c, 0)),
            pl.BlockSpec((CA, 1), lambda c: (c, 0)),
            pl.BlockSpec((CA, LANES), lambda c: (c, 0)),
            pl.BlockSpec((CA, LANES), lambda c: (c, 0)),
            pl.BlockSpec((1, 1, EPAD), lambda c: (c, 0, 0)),
            pl.BlockSpec((1, 1, EPAD), lambda c: (c, 0, 0)),
            pl.BlockSpec((CA, 1), lambda c: (c, 0)),
            pl.BlockSpec((CA, 1), lambda c: (c, 0)),
        ],
        out_shape=[
            jax.ShapeDtypeStruct((T, 1), jnp.int32),
            jax.ShapeDtypeStruct((T, 1), jnp.int32),
            jax.ShapeDtypeStruct((T, LANES), jnp.float32),
            jax.ShapeDtypeStruct((T, LANES), jnp.float32),
            jax.ShapeDtypeStruct((NCA, 1, EPAD), jnp.float32),
            jax.ShapeDtypeStruct((NCA, 1, EPAD), jnp.float32),
            jax.ShapeDtypeStruct((T, 1), jnp.float32),
            jax.ShapeDtypeStruct((T, 1), jnp.float32),
        ],
    )(x, wg_pad, steer_pad)

    s0, s1, bexp = pl.pallas_call(
        _k1c_body,
        out_shape=[
            jax.ShapeDtypeStruct((T, 1), jnp.int32),
            jax.ShapeDtypeStruct((T, 1), jnp.int32),
            jax.ShapeDtypeStruct((NB + 1, 1), jnp.int32),
        ],
    )(a1, a2, p1, p2, c1.reshape(NCA, EPAD), c2.reshape(NCA, EPAD))

    xs = _k2_sc(s0.reshape(T), s1.reshape(T), x)

    y = pl.pallas_call(
        _k3_body,
        grid_spec=pltpu.PrefetchScalarGridSpec(
            num_scalar_prefetch=1,
            grid=(NB,),
            in_specs=[
                pl.BlockSpec((B, D_MODEL), lambda b, be: (b, 0)),
                pl.BlockSpec((1, D_MODEL, D_FF), lambda b, be: (be[b], 0, 0)),
                pl.BlockSpec((1, D_MODEL, D_FF), lambda b, be: (be[b], 0, 0)),
                pl.BlockSpec((1, D_FF, D_MODEL), lambda b, be: (be[b], 0, 0)),
            ],
            out_specs=pl.BlockSpec((B, D_MODEL), lambda b, be: (b, 0)),
        ),
        out_shape=jax.ShapeDtypeStruct((S, D_MODEL), jnp.float32),
        compiler_params=pltpu.CompilerParams(
            dimension_semantics=("arbitrary",),
        ),
    )(bexp.reshape(NB + 1), xs, W1, W3, W2)

    return _k4_sc(y, s0.reshape(T), s1.reshape(T), wb0, wb1)
